# baseline (speedup 1.0000x reference)
import functools

import jax
import jax.numpy as jnp
from jax import lax
from jax.experimental import pallas as pl
from jax.experimental.pallas import tpu as pltpu
from jax.experimental.pallas import tpu_sc as plsc

HIDDEN = 768
EPS = 1e-5
L = 16
NVREG = HIDDEN // L
NC = 2
NS = 16
NW = NC * NS
B_TOTAL = 4 * 8192
B_PER_W = B_TOTAL // NW
CHUNK = 64
N_CHUNKS = B_PER_W // CHUNK


def _allreduce_sum(x, perm_idx):
    for idx in perm_idx:
        x = x + x.at[idx].get(mode="promise_in_bounds")
    return x


def _rsqrt_vec(x):
    i = lax.bitcast_convert_type(x, jnp.int32)
    i = jnp.int32(0x5F3759DF) - lax.shift_right_arithmetic(i, 1)
    y = lax.bitcast_convert_type(i, jnp.float32)
    for _ in range(3):
        y = y * (1.5 - 0.5 * x * y * y)
    return y


def kernel(input_ids, tok_embeddings, norm_weight):
    ids_flat = input_ids.reshape(B_TOTAL)
    mesh = plsc.VectorSubcoreMesh(core_axis_name="c", subcore_axis_name="s")

    @functools.partial(
        pl.kernel,
        mesh=mesh,
        out_type=jax.ShapeDtypeStruct((B_TOTAL, HIDDEN), jnp.float32),
        scratch_types=[
            pltpu.VMEM((CHUNK,), jnp.int32),
            pltpu.VMEM((CHUNK, HIDDEN), jnp.float32),
            pltpu.VMEM((HIDDEN,), jnp.float32),
            pltpu.SemaphoreType.DMA,
        ],
    )
    def run(table_hbm, idx_hbm, w_hbm, out_hbm, idx_v, rows_v, w_v, sem):
        wid = lax.axis_index("s") * NC + lax.axis_index("c")
        pltpu.sync_copy(w_hbm, w_v)
        lanes = lax.iota(jnp.int32, L)
        perm_idx = [lanes ^ sh for sh in (8, 4, 2, 1)]

        def chunk_body(g, _):
            base = wid * B_PER_W + g * CHUNK
            pltpu.sync_copy(idx_hbm.at[pl.ds(base, CHUNK)], idx_v)
            pltpu.async_copy(table_hbm.at[idx_v], rows_v, sem).wait()

            def row_body(r, _):
                s = jnp.zeros((L,), jnp.float32)
                q = jnp.zeros((L,), jnp.float32)
                for j in range(NVREG):
                    x = rows_v[r, pl.ds(j * L, L)]
                    s = s + x
                    q = q + x * x
                mean_v = _allreduce_sum(s, perm_idx) * (1.0 / HIDDEN)
                m2_v = _allreduce_sum(q, perm_idx) * (1.0 / HIDDEN)
                var_v = m2_v - mean_v * mean_v
                inv = _rsqrt_vec(var_v + EPS)
                for j in range(NVREG):
                    x = rows_v[r, pl.ds(j * L, L)]
                    y = (x - mean_v) * inv * w_v[pl.ds(j * L, L)]
                    rows_v[r, pl.ds(j * L, L)] = y
                return 0

            lax.fori_loop(0, CHUNK, row_body, 0)
            pltpu.sync_copy(rows_v, out_hbm.at[pl.ds(base, CHUNK)])
            return 0

        lax.fori_loop(0, N_CHUNKS, chunk_body, 0)

    out = run(tok_embeddings, ids_flat, norm_weight)
    return out.reshape(input_ids.shape[0], input_ids.shape[1], HIDDEN)


# speedup vs baseline: 1.1193x; 1.1193x over previous
import functools

import jax
import jax.numpy as jnp
from jax import lax
from jax.experimental import pallas as pl
from jax.experimental.pallas import tpu as pltpu
from jax.experimental.pallas import tpu_sc as plsc

HIDDEN = 768
EPS = 1e-5
L = 16
NVREG = HIDDEN // L
NC = 2
NS = 16
NW = NC * NS
B_TOTAL = 4 * 8192
B_PER_W = B_TOTAL // NW
CHUNK = 64
N_CHUNKS = B_PER_W // CHUNK


def _allreduce_sum(x, perm_idx):
    for idx in perm_idx:
        x = x + x.at[idx].get(mode="promise_in_bounds")
    return x


def _rsqrt_vec(x):
    i = lax.bitcast_convert_type(x, jnp.int32)
    i = jnp.int32(0x5F3759DF) - lax.shift_right_arithmetic(i, 1)
    y = lax.bitcast_convert_type(i, jnp.float32)
    for _ in range(3):
        y = y * (1.5 - 0.5 * x * y * y)
    return y


def kernel(input_ids, tok_embeddings, norm_weight):
    ids_flat = input_ids.reshape(B_TOTAL)
    mesh = plsc.VectorSubcoreMesh(core_axis_name="c", subcore_axis_name="s")

    @functools.partial(
        pl.kernel,
        mesh=mesh,
        out_type=jax.ShapeDtypeStruct((B_TOTAL, HIDDEN), jnp.float32),
        scratch_types=[
            pltpu.VMEM((B_PER_W,), jnp.int32),
            pltpu.VMEM((CHUNK, HIDDEN), jnp.float32),
            pltpu.VMEM((CHUNK, HIDDEN), jnp.float32),
            pltpu.VMEM((HIDDEN,), jnp.float32),
            pltpu.SemaphoreType.DMA,
            pltpu.SemaphoreType.DMA,
            pltpu.SemaphoreType.DMA,
            pltpu.SemaphoreType.DMA,
        ],
    )
    def run(table_hbm, idx_hbm, w_hbm, out_hbm,
            idx_all, rows0, rows1, w_v,
            sem_in0, sem_in1, sem_out0, sem_out1):
        wid = lax.axis_index("s") * NC + lax.axis_index("c")
        base_w = wid * B_PER_W
        pltpu.sync_copy(w_hbm, w_v)
        pltpu.sync_copy(idx_hbm.at[pl.ds(base_w, B_PER_W)], idx_all)
        lanes = lax.iota(jnp.int32, L)
        perm_idx = [lanes ^ sh for sh in (8, 4, 2, 1)]

        rows = (rows0, rows1)
        sem_in = (sem_in0, sem_in1)
        sem_out = (sem_out0, sem_out1)

        def gather_start(g, b):
            pltpu.async_copy(
                table_hbm.at[idx_all.at[pl.ds(g * CHUNK, CHUNK)]],
                rows[b], sem_in[b])

        def gather_wait(b):
            pltpu.make_async_copy(
                table_hbm.at[idx_all.at[pl.ds(0, CHUNK)]],
                rows[b], sem_in[b]).wait()

        def out_start(g, b):
            pltpu.async_copy(
                rows[b], out_hbm.at[pl.ds(base_w + g * CHUNK, CHUNK)],
                sem_out[b])

        def out_wait(b):
            pltpu.make_async_copy(
                rows[b], out_hbm.at[pl.ds(0, CHUNK)], sem_out[b]).wait()

        def compute(b):
            rv = rows[b]

            def row_body(r, _):
                s = jnp.zeros((L,), jnp.float32)
                q = jnp.zeros((L,), jnp.float32)
                for j in range(NVREG):
                    x = rv[r, pl.ds(j * L, L)]
                    s = s + x
                    q = q + x * x
                mean_v = _allreduce_sum(s, perm_idx) * (1.0 / HIDDEN)
                m2_v = _allreduce_sum(q, perm_idx) * (1.0 / HIDDEN)
                var_v = m2_v - mean_v * mean_v
                inv = _rsqrt_vec(var_v + EPS)
                for j in range(NVREG):
                    x = rv[r, pl.ds(j * L, L)]
                    y = (x - mean_v) * inv * w_v[pl.ds(j * L, L)]
                    rv[r, pl.ds(j * L, L)] = y
                return 0

            lax.fori_loop(0, CHUNK, row_body, 0)

        gather_start(0, 0)

        gather_wait(0)
        gather_start(1, 1)
        compute(0)
        out_start(0, 0)

        def steady(t, _):
            g1 = 1 + 2 * t
            gather_wait(1)
            out_wait(0)
            gather_start(g1 + 1, 0)
            compute(1)
            out_start(g1, 1)

            gather_wait(0)
            out_wait(1)
            gather_start(g1 + 2, 1)
            compute(0)
            out_start(g1 + 1, 0)
            return 0

        lax.fori_loop(0, (N_CHUNKS - 2) // 2, steady, 0)

        gather_wait(1)
        compute(1)
        out_start(N_CHUNKS - 1, 1)

        out_wait(0)
        out_wait(1)

    out = run(tok_embeddings, ids_flat, norm_weight)
    return out.reshape(input_ids.shape[0], input_ids.shape[1], HIDDEN)


# speedup vs baseline: 1.7550x; 1.5679x over previous
import functools

import jax
import jax.numpy as jnp
from jax import lax
from jax.experimental import pallas as pl
from jax.experimental.pallas import tpu as pltpu
from jax.experimental.pallas import tpu_sc as plsc

HIDDEN = 768
EPS = 1e-5
L = 16
NVREG = HIDDEN // L
NC = 2
NS = 16
NW = NC * NS
B_TOTAL = 4 * 8192
B_PER_W = B_TOTAL // NW
CHUNK = 64
N_CHUNKS = B_PER_W // CHUNK


def _allreduce_sum(x, perm_idx):
    for idx in perm_idx:
        x = x + x.at[idx].get(mode="promise_in_bounds")
    return x


def _rsqrt_vec(x):
    i = lax.bitcast_convert_type(x, jnp.int32)
    i = jnp.int32(0x5F3759DF) - lax.shift_right_arithmetic(i, 1)
    y = lax.bitcast_convert_type(i, jnp.float32)
    for _ in range(3):
        y = y * (1.5 - 0.5 * x * y * y)
    return y


def kernel(input_ids, tok_embeddings, norm_weight):
    ids_flat = input_ids.reshape(B_TOTAL)
    mesh = plsc.VectorSubcoreMesh(core_axis_name="c", subcore_axis_name="s")

    @functools.partial(
        pl.kernel,
        mesh=mesh,
        out_type=jax.ShapeDtypeStruct((B_TOTAL, HIDDEN), jnp.float32),
        scratch_types=[
            pltpu.VMEM((B_PER_W,), jnp.int32),
            pltpu.VMEM((CHUNK, HIDDEN), jnp.float32),
            pltpu.VMEM((CHUNK, HIDDEN), jnp.float32),
            pltpu.VMEM((HIDDEN,), jnp.float32),
            pltpu.SemaphoreType.DMA,
            pltpu.SemaphoreType.DMA,
            pltpu.SemaphoreType.DMA,
            pltpu.SemaphoreType.DMA,
        ],
    )
    def run(table_hbm, idx_hbm, w_hbm, out_hbm,
            idx_all, rows0, rows1, w_v,
            sem_in0, sem_in1, sem_out0, sem_out1):
        wid = lax.axis_index("s") * NC + lax.axis_index("c")
        base_w = wid * B_PER_W
        pltpu.sync_copy(w_hbm, w_v)
        pltpu.sync_copy(idx_hbm.at[pl.ds(base_w, B_PER_W)], idx_all)
        lanes = lax.iota(jnp.int32, L)
        perm_idx = [lanes ^ sh for sh in (8, 4, 2, 1)]

        rows = (rows0, rows1)
        sem_in = (sem_in0, sem_in1)
        sem_out = (sem_out0, sem_out1)

        def gather_start(g, b):
            pltpu.async_copy(
                table_hbm.at[idx_all.at[pl.ds(g * CHUNK, CHUNK)]],
                rows[b], sem_in[b])

        def gather_wait(b):
            pltpu.make_async_copy(
                table_hbm.at[idx_all.at[pl.ds(0, CHUNK)]],
                rows[b], sem_in[b]).wait()

        def out_start(g, b):
            pltpu.async_copy(
                rows[b], out_hbm.at[pl.ds(base_w + g * CHUNK, CHUNK)],
                sem_out[b])

        def out_wait(b):
            pltpu.make_async_copy(
                rows[b], out_hbm.at[pl.ds(0, CHUNK)], sem_out[b]).wait()

        def compute(b):
            rv = rows[b]

            def row_stats(r):
                s = [jnp.zeros((L,), jnp.float32) for _ in range(4)]
                q = [jnp.zeros((L,), jnp.float32) for _ in range(4)]
                for j in range(NVREG):
                    x = rv[r, pl.ds(j * L, L)]
                    k = j % 4
                    s[k] = s[k] + x
                    q[k] = q[k] + x * x
                st = (s[0] + s[1]) + (s[2] + s[3])
                qt = (q[0] + q[1]) + (q[2] + q[3])
                mean_v = _allreduce_sum(st, perm_idx) * (1.0 / HIDDEN)
                m2_v = _allreduce_sum(qt, perm_idx) * (1.0 / HIDDEN)
                var_v = m2_v - mean_v * mean_v
                return mean_v, _rsqrt_vec(var_v + EPS)

            def pair_body(rr, _):
                ra = rr * 2
                rb = ra + 1
                mean_a, inv_a = row_stats(ra)
                mean_b, inv_b = row_stats(rb)
                for j in range(NVREG):
                    w = w_v[pl.ds(j * L, L)]
                    xa = rv[ra, pl.ds(j * L, L)]
                    xb = rv[rb, pl.ds(j * L, L)]
                    rv[ra, pl.ds(j * L, L)] = (xa - mean_a) * inv_a * w
                    rv[rb, pl.ds(j * L, L)] = (xb - mean_b) * inv_b * w
                return 0

            lax.fori_loop(0, CHUNK // 2, pair_body, 0)

        gather_start(0, 0)

        gather_wait(0)
        gather_start(1, 1)
        compute(0)
        out_start(0, 0)

        def steady(t, _):
            g1 = 1 + 2 * t
            gather_wait(1)
            out_wait(0)
            gather_start(g1 + 1, 0)
            compute(1)
            out_start(g1, 1)

            gather_wait(0)
            out_wait(1)
            gather_start(g1 + 2, 1)
            compute(0)
            out_start(g1 + 1, 0)
            return 0

        lax.fori_loop(0, (N_CHUNKS - 2) // 2, steady, 0)

        gather_wait(1)
        compute(1)
        out_start(N_CHUNKS - 1, 1)

        out_wait(0)
        out_wait(1)

    out = run(tok_embeddings, ids_flat, norm_weight)
    return out.reshape(input_ids.shape[0], input_ids.shape[1], HIDDEN)
